# baseline (speedup 1.0000x reference)
import functools

import jax
import jax.numpy as jnp
from jax import lax
from jax.experimental import pallas as pl
from jax.experimental.pallas import tpu as pltpu
from jax.experimental.pallas import tpu_sc as plsc

VOCAB = 1000
D = 1000
DA = 896
DB = D - DA
B, T = 4096, 50
TP = 56
NC, NS = 2, 16
NW = NC * NS
B_PER_W = B // NW


def _sc_gather(table_a, table_b, idx3):
    mesh = plsc.VectorSubcoreMesh(core_axis_name="c", subcore_axis_name="s")

    @functools.partial(
        pl.kernel,
        mesh=mesh,
        out_type=jax.ShapeDtypeStruct((B * TP, D), jnp.float32),
        scratch_types=[
            pltpu.VMEM((B_PER_W, TP), jnp.int32),
            pltpu.VMEM((TP, DA), jnp.float32),
            pltpu.VMEM((TP, DA), jnp.float32),
            pltpu.VMEM((TP, 128), jnp.float32),
            pltpu.VMEM((TP, DB), jnp.float32),
            pltpu.SemaphoreType.DMA,
            pltpu.SemaphoreType.DMA,
            pltpu.SemaphoreType.DMA,
            pltpu.SemaphoreType.DMA,
            pltpu.SemaphoreType.DMA,
            pltpu.SemaphoreType.DMA,
        ],
    )
    def k(ta_hbm, tb_hbm, idx_hbm, out_hbm, idx_v,
          bufa0, bufa1, bufb, bufc, ga0, ga1, gb, wa0, wa1, wb):
        sid = lax.axis_index("s")
        wid = sid * NC + lax.axis_index("c")
        base = wid * B_PER_W
        pltpu.sync_copy(idx_hbm.at[wid], idx_v)

        bufas = (bufa0, bufa1)
        gasems = (ga0, ga1)
        wasems = (wa0, wa1)

        def start_ga(j, p):
            pltpu.async_copy(ta_hbm.at[idx_v.at[j]], bufas[p], gasems[p])

        def wait_ga(j, p):
            pltpu.make_async_copy(
                ta_hbm.at[idx_v.at[j]], bufas[p], gasems[p]).wait()

        def start_gb(j):
            pltpu.async_copy(tb_hbm.at[idx_v.at[j]], bufb, gb)

        def wait_gb(j):
            pltpu.make_async_copy(tb_hbm.at[idx_v.at[j]], bufb, gb).wait()

        def compact_tail():
            def row_copy(r, carry):
                for c in range(0, 96, 16):
                    bufc[r, pl.ds(c, 16)] = bufb[r, pl.ds(c, 16)]
                bufc[r, pl.ds(88, 16)] = bufb[r, pl.ds(88, 16)]
                return carry

            lax.fori_loop(0, TP, row_copy, 0)

        def start_wa(j, p):
            pltpu.async_copy(
                bufas[p],
                out_hbm.at[pl.ds((base + j) * TP, TP), pl.ds(0, DA)],
                wasems[p])

        def wait_wa(j, p):
            pltpu.make_async_copy(
                bufas[p],
                out_hbm.at[pl.ds((base + j) * TP, TP), pl.ds(0, DA)],
                wasems[p]).wait()

        def start_wb(j):
            pltpu.async_copy(
                bufc,
                out_hbm.at[pl.ds((base + j) * TP, TP), pl.ds(DA, DB)], wb)

        def wait_wb(j):
            pltpu.make_async_copy(
                bufc,
                out_hbm.at[pl.ds((base + j) * TP, TP), pl.ds(DA, DB)],
                wb).wait()

        def step(j, p):
            wait_ga(j, p)
            wait_gb(j)
            compact_tail()
            start_wa(j, p)
            start_wb(j)
            wait_wa(j - 1, 1 - p)
            start_ga(j + 1, 1 - p)
            wait_wb(j)
            start_gb(j + 1)

        start_ga(0, 0)
        start_gb(0)
        wait_ga(0, 0)
        wait_gb(0)
        compact_tail()
        start_wa(0, 0)
        start_wb(0)
        start_ga(1, 1)
        wait_wb(0)
        start_gb(1)

        def body(jj, carry):
            step(2 * jj + 1, 1)
            step(2 * jj + 2, 0)
            return carry

        lax.fori_loop(0, (B_PER_W - 4) // 2, body, 0)

        step(B_PER_W - 3, 1)
        step(B_PER_W - 2, 0)
        j_last = B_PER_W - 1
        wait_ga(j_last, 1)
        wait_gb(j_last)
        compact_tail()
        start_wa(j_last, 1)
        start_wb(j_last)
        wait_wa(j_last - 1, 0)
        wait_wa(j_last, 1)
        wait_wb(j_last)

    return k(table_a, table_b, idx3)


def kernel(idx, table):
    idxp = jnp.pad(idx.astype(jnp.int32), ((0, 0), (0, TP - T)))
    idx3 = idxp.reshape(NW, B_PER_W, TP)
    table_a = table[:, :DA]
    table_b = jnp.pad(table[:, DA:], ((0, 0), (0, 128 - DB)))
    out = _sc_gather(table_a, table_b, idx3)
    return out.reshape(B, TP, D)[:, :T, :]
